# baseline (speedup 1.0000x reference)
import numpy as np
import jax
import jax.numpy as jnp
from jax.experimental import pallas as pl
from jax.experimental.pallas import tpu as pltpu

_BN = 1024


def _band(src, half, par):
    a = np.zeros((src, half, 5), np.float32)
    for w in range(half):
        for e in range(5):
            a[2 * w + par + e, w, e] = 1.0
    return a


_A1 = (_band(32, 14, 0), _band(32, 14, 1))
_A2 = (_band(14, 5, 0), _band(14, 5, 1))

_P2PERM = np.arange(400).reshape(16, 5, 5).transpose(1, 2, 0).reshape(400)


def _lenet_block(x_ref, s1a_ref, s1b_ref, b1_ref, s2a_ref, s2b_ref, b2_ref,
                 w3_ref, b3_ref, w4_ref, b4_ref, w5_ref, b5_ref,
                 o_ref, m1_s, m2_s, p1_s, p2_s):
    f32 = jnp.float32
    bf16 = jnp.bfloat16

    @pl.when(pl.program_id(1) == 0)
    def _build():
        m1_s[...] = jnp.zeros((256, 672), bf16)
        m2_s[...] = jnp.zeros((768, 320), bf16)
        p1_s[...] = jnp.zeros(p1_s.shape, bf16)
        s1 = (s1a_ref[...], s1b_ref[...])
        s2 = (s2a_ref[...], s2b_ref[...])
        for py in (0, 1):
            for px in (0, 1):
                g = 2 * py + px
                for hh in (0, 1):
                    r = 64 * hh + 32 * py
                    c = g * 168 + 84 * hh
                    m1_s[r:r + 160, c:c + 84] = s1[px]
                for d in range(5):
                    s = py + d
                    r = 256 * (s // 2) + 84 * (s % 2)
                    m2_s[r:r + 84, g * 80:g * 80 + 80] = \
                        s2[px][84 * d:84 * d + 84, :]

    xv = x_ref[...]
    m1v = m1_s[...]
    b1v = b1_ref[...]
    for p in range(7):
        wp = xv[:, 4 * p:4 * p + 8, :].reshape(xv.shape[0], 256)
        y = jnp.dot(wp, m1v,
                    preferred_element_type=f32)
        q = jnp.maximum(jnp.maximum(y[:, 0:168], y[:, 168:336]),
                        jnp.maximum(y[:, 336:504], y[:, 504:672]))
        p1_s[:, 256 * p:256 * p + 168] = \
            jnp.maximum(q + b1v, 0.0).astype(bf16)

    m2v = m2_s[...]
    b2v = b2_ref[...]
    for h in range(5):
        y = jnp.dot(p1_s[:, 256 * h:256 * h + 768], m2v,
                    preferred_element_type=f32)
        q = jnp.maximum(jnp.maximum(y[:, 0:80], y[:, 80:160]),
                        jnp.maximum(y[:, 160:240], y[:, 240:320]))
        p2_s[:, 80 * h:80 * h + 80] = \
            jnp.maximum(q + b2v, 0.0).astype(bf16)

    h1 = jnp.maximum(jnp.dot(p2_s[...], w3_ref[...],
                             preferred_element_type=f32) + b3_ref[...], 0.0)
    h2 = jnp.maximum(jnp.dot(h1.astype(bf16), w4_ref[...],
                             preferred_element_type=f32) + b4_ref[...], 0.0)
    o_ref[...] = jnp.dot(h2.astype(bf16), w5_ref[...],
                         preferred_element_type=f32) + b5_ref[...]


@jax.jit
def kernel(x, conv1_w, conv1_b, conv2_w, conv2_b,
           fc1_w, fc1_b, fc2_w, fc2_b, fc3_w, fc3_b):
    bf16 = jnp.bfloat16
    B = x.shape[0]
    x3 = x.astype(bf16).reshape(B, 32, 32)

    w1b = conv1_w.reshape(6, 5, 5).astype(bf16)
    s1 = [jnp.einsum('kde,xwe->dxwk', w1b, jnp.asarray(_A1[px], bf16)
                     ).reshape(160, 84) for px in (0, 1)]
    b1 = jnp.broadcast_to(conv1_b[None, :], (28, 6)).reshape(1, 168)

    w2b = conv2_w.astype(bf16)
    s2 = [jnp.einsum('kcde,xwe->dxcwk', w2b, jnp.asarray(_A2[px], bf16)
                     ).reshape(420, 80) for px in (0, 1)]
    b2 = jnp.broadcast_to(conv2_b[None, :], (5, 16)).reshape(1, 80)

    w3 = fc1_w[:, _P2PERM].T.astype(bf16)
    w4 = fc2_w.T.astype(bf16)
    w5 = fc3_w.T.astype(bf16)
    b3 = fc1_b.reshape(1, 120)
    b4 = fc2_b.reshape(1, 84)
    b5 = fc3_b.reshape(1, 10)

    pad = (-B) % (2 * _BN)
    if pad:
        x3 = jnp.pad(x3, ((0, pad), (0, 0), (0, 0)))
    bp = B + pad
    inner = bp // _BN // 2

    def const(a):
        return pl.BlockSpec(a.shape, lambda i, j, _nd=a.ndim: (0,) * _nd)

    out = pl.pallas_call(
        _lenet_block,
        out_shape=jax.ShapeDtypeStruct((bp, 10), jnp.float32),
        grid=(2, inner),
        in_specs=[
            pl.BlockSpec((_BN, 32, 32), lambda i, j, _n=inner: (i * _n + j, 0, 0)),
            const(s1[0]), const(s1[1]), const(b1),
            const(s2[0]), const(s2[1]), const(b2),
            const(w3), const(b3), const(w4), const(b4), const(w5), const(b5),
        ],
        out_specs=pl.BlockSpec((_BN, 10),
                               lambda i, j, _n=inner: (i * _n + j, 0)),
        scratch_shapes=[pltpu.VMEM((256, 672), bf16),
                        pltpu.VMEM((768, 320), bf16),
                        pltpu.VMEM((_BN, 7 * 256), bf16),
                        pltpu.VMEM((_BN, 400), bf16)],
        compiler_params=pltpu.CompilerParams(
            dimension_semantics=("parallel", "arbitrary")),
    )(x3, s1[0], s1[1], b1, s2[0], s2[1], b2,
      w3, b3, w4, b4, w5, b5)
    return out[:B] if pad else out


# speedup vs baseline: 1.9501x; 1.9501x over previous
import numpy as np
import jax
import jax.numpy as jnp
from jax.experimental import pallas as pl
from jax.experimental.pallas import tpu as pltpu

_BN = 2048


def _band(src, half, par):
    a = np.zeros((src, half, 5), np.float32)
    for w in range(half):
        for e in range(5):
            a[2 * w + par + e, w, e] = 1.0
    return a


_A1 = (_band(32, 14, 0), _band(32, 14, 1))
_A2 = (_band(14, 5, 0), _band(14, 5, 1))

_P2PERM = np.arange(400).reshape(16, 5, 5).transpose(1, 2, 0).reshape(400)


def _lenet_block(x_ref, s1a_ref, s1b_ref, b1_ref, s2a_ref, s2b_ref, b2_ref,
                 w3_ref, b3_ref, w4_ref, b4_ref, w5_ref, b5_ref,
                 o_ref, m1_s, m2_s, p1_s, p2_s):
    f32 = jnp.float32
    bf16 = jnp.bfloat16

    @pl.when(pl.program_id(1) == 0)
    def _build():
        m1_s[...] = jnp.zeros((256, 672), bf16)
        m2_s[...] = jnp.zeros((768, 320), bf16)
        p1_s[...] = jnp.zeros(p1_s.shape, bf16)
        s1 = (s1a_ref[...], s1b_ref[...])
        s2 = (s2a_ref[...], s2b_ref[...])
        for py in (0, 1):
            for px in (0, 1):
                g = 2 * py + px
                for hh in (0, 1):
                    r = 64 * hh + 32 * py
                    c = g * 168 + 84 * hh
                    m1_s[r:r + 160, c:c + 84] = s1[px]
                for d in range(5):
                    s = py + d
                    r = 256 * (s // 2) + 84 * (s % 2)
                    m2_s[r:r + 84, g * 80:g * 80 + 80] = \
                        s2[px][84 * d:84 * d + 84, :]

    xb = x_ref[...]
    m1v = m1_s[...]
    b1v = b1_ref[...]
    for p in range(7):
        y = jnp.dot(xb[:, 128 * p:128 * p + 256], m1v,
                    preferred_element_type=f32)
        q = jnp.maximum(jnp.maximum(y[:, 0:168], y[:, 168:336]),
                        jnp.maximum(y[:, 336:504], y[:, 504:672]))
        p1_s[:, 256 * p:256 * p + 168] = \
            jnp.maximum(q + b1v, 0.0).astype(bf16)

    m2v = m2_s[...]
    b2v = b2_ref[...]
    for h in range(5):
        y = jnp.dot(p1_s[:, 256 * h:256 * h + 768], m2v,
                    preferred_element_type=f32)
        q = jnp.maximum(jnp.maximum(y[:, 0:80], y[:, 80:160]),
                        jnp.maximum(y[:, 160:240], y[:, 240:320]))
        p2_s[:, 80 * h:80 * h + 80] = \
            jnp.maximum(q + b2v, 0.0).astype(bf16)

    h1 = jnp.maximum(jnp.dot(p2_s[...], w3_ref[...],
                             preferred_element_type=f32) + b3_ref[...], 0.0)
    h2 = jnp.maximum(jnp.dot(h1.astype(bf16), w4_ref[...],
                             preferred_element_type=f32) + b4_ref[...], 0.0)
    o_ref[...] = jnp.dot(h2.astype(bf16), w5_ref[...],
                         preferred_element_type=f32) + b5_ref[...]


@jax.jit
def kernel(x, conv1_w, conv1_b, conv2_w, conv2_b,
           fc1_w, fc1_b, fc2_w, fc2_b, fc3_w, fc3_b):
    bf16 = jnp.bfloat16
    B = x.shape[0]
    x2d = x.astype(bf16).reshape(B, 32 * 32)

    w1b = conv1_w.reshape(6, 5, 5).astype(bf16)
    s1 = [jnp.einsum('kde,xwe->dxwk', w1b, jnp.asarray(_A1[px], bf16)
                     ).reshape(160, 84) for px in (0, 1)]
    b1 = jnp.broadcast_to(conv1_b[None, :], (28, 6)).reshape(1, 168)

    w2b = conv2_w.astype(bf16)
    s2 = [jnp.einsum('kcde,xwe->dxcwk', w2b, jnp.asarray(_A2[px], bf16)
                     ).reshape(420, 80) for px in (0, 1)]
    b2 = jnp.broadcast_to(conv2_b[None, :], (5, 16)).reshape(1, 80)

    w3 = fc1_w[:, _P2PERM].T.astype(bf16)
    w4 = fc2_w.T.astype(bf16)
    w5 = fc3_w.T.astype(bf16)
    b3 = fc1_b.reshape(1, 120)
    b4 = fc2_b.reshape(1, 84)
    b5 = fc3_b.reshape(1, 10)

    pad = (-B) % (2 * _BN)
    if pad:
        x2d = jnp.pad(x2d, ((0, pad), (0, 0)))
    bp = B + pad
    inner = bp // _BN // 2

    def const(a):
        return pl.BlockSpec(a.shape, lambda i, j, _nd=a.ndim: (0,) * _nd)

    out = pl.pallas_call(
        _lenet_block,
        out_shape=jax.ShapeDtypeStruct((bp, 10), jnp.float32),
        grid=(2, inner),
        in_specs=[
            pl.BlockSpec((_BN, 1024), lambda i, j, _n=inner: (i * _n + j, 0)),
            const(s1[0]), const(s1[1]), const(b1),
            const(s2[0]), const(s2[1]), const(b2),
            const(w3), const(b3), const(w4), const(b4), const(w5), const(b5),
        ],
        out_specs=pl.BlockSpec((_BN, 10),
                               lambda i, j, _n=inner: (i * _n + j, 0)),
        scratch_shapes=[pltpu.VMEM((256, 672), bf16),
                        pltpu.VMEM((768, 320), bf16),
                        pltpu.VMEM((_BN, 7 * 256), bf16),
                        pltpu.VMEM((_BN, 400), bf16)],
        compiler_params=pltpu.CompilerParams(
            dimension_semantics=("parallel", "arbitrary")),
    )(x2d, s1[0], s1[1], b1, s2[0], s2[1], b2,
      w3, b3, w4, b4, w5, b5)
    return out[:B] if pad else out
